# baseline (speedup 1.0000x reference)
import jax
import jax.numpy as jnp
from jax.experimental import pallas as pl
from jax.experimental.pallas import tpu as pltpu

E = 2048
N = 1024
OUT = 2 * E + N
DT = 1e-06

R = 512
C = OUT // 2
NB = OUT // R
N_KCL = N // R
N_KVL = E // R
N_EL = E // R


def _band_kernel(m_hbm, p_ref, k_ref, s_ref, out_ref, m_vmem, zy_vmem, sem):
    i = pl.program_id(0)
    j = pl.program_id(1)

    @pl.when(jnp.logical_and(i == 0, j == 0))
    def _first_step():
        pltpu.make_async_copy(m_hbm, m_vmem, sem).start()
        params = p_ref[...]
        kinds = k_ref[...]
        sw_on = s_ref[...] > 0.0
        z = jnp.where(kinds == 0, -params,
            jnp.where(kinds == 4, -DT / params,
            jnp.where(kinds == 5, 1.0,
            jnp.where(kinds == 2, 1.0,
            jnp.where(jnp.logical_and(kinds == 3, jnp.logical_not(sw_on)),
                      1.0, 0.0)))))
        y = jnp.where(kinds == 0, 1.0,
            jnp.where(kinds == 4, 1.0,
            jnp.where(kinds == 5, -DT / params,
            jnp.where(kinds == 1, 1.0,
            jnp.where(jnp.logical_and(kinds == 3, sw_on), 1.0, 0.0)))))
        zy_vmem[:, 0:E] = z
        zy_vmem[:, E:2 * E] = y
        zy_vmem[:, 2 * E:] = jnp.zeros((1, N), jnp.float32)

    @pl.when(jnp.logical_and(i == N_EL, j == 1))
    def _wait_m_copy():
        pltpu.make_async_copy(m_hbm, m_vmem, sem).wait()

    def _zero_cols(lo, hi):
        if hi > lo:
            out_ref[:, lo:hi] = jnp.zeros((R, hi - lo), jnp.float32)

    def _strip(a, value):
        rs = jax.lax.broadcasted_iota(jnp.int32, (R, R), 0)
        cs = jax.lax.broadcasted_iota(jnp.int32, (R, R), 1)
        out_ref[:, a:a + R] = jnp.where(cs == rs, value, 0.0)

    def _diag_tile(c0, diags, m_e0=None):
        hi = 2 * E - c0 if m_e0 is not None else C
        strips = sorted((d - c0, v) for d, v in diags
                        if 0 <= d - c0 and d - c0 + R <= hi)
        pos = 0
        for a, v in strips:
            _zero_cols(pos, a)
            _strip(a, v)
            pos = a + R
        _zero_cols(pos, hi)
        if m_e0 is not None:
            out_ref[:, hi:] = -m_vmem[:, m_e0:m_e0 + R].T

    def _el(e0, c0):
        _diag_tile(c0, [(e0, zy_vmem[0:1, e0:e0 + R]),
                        (E + e0, zy_vmem[0:1, E + e0:E + e0 + R])])

    def _kvl(e0, c0):
        _diag_tile(c0, [(E + e0, 1.0)], m_e0=e0 if c0 + C > 2 * E else None)

    def _kcl(r0, c0):
        if c0 == 0:
            out_ref[:, 0:E] = m_vmem[r0:r0 + R, :]
            _zero_cols(E, C)
        else:
            _zero_cols(0, C)

    for b in range(N_EL):
        for jj in range(2):
            pl.when(jnp.logical_and(i == b, j == jj))(
                lambda b=b, jj=jj: _el(b * R, jj * C))
    for b in range(N_KVL):
        for jj in range(2):
            pl.when(jnp.logical_and(i == N_EL + b, j == jj))(
                lambda b=b, jj=jj: _kvl(b * R, jj * C))
    for b in range(N_KCL):
        for jj in range(2):
            pl.when(jnp.logical_and(i == N_EL + N_KVL + b, j == jj))(
                lambda b=b, jj=jj: _kcl(b * R, jj * C))


def _out_band(s):
    return jnp.where(s < N_EL, s + N_KCL + N_KVL,
           jnp.where(s < N_EL + N_KVL, s - N_EL + N_KCL,
                     s - N_EL - N_KVL))


def kernel(M, params, sw_params, kinds, time):
    swcol = sw_params[:, time]
    p2 = params.reshape(1, E).astype(jnp.float32)
    k2 = kinds.reshape(1, E).astype(jnp.int32)
    s2 = swcol.reshape(1, E).astype(jnp.float32)

    out = pl.pallas_call(
        _band_kernel,
        grid=(NB, 2),
        in_specs=[
            pl.BlockSpec(memory_space=pl.ANY),
            pl.BlockSpec((1, E), lambda i, j: (0, 0)),
            pl.BlockSpec((1, E), lambda i, j: (0, 0)),
            pl.BlockSpec((1, E), lambda i, j: (0, 0)),
        ],
        out_specs=pl.BlockSpec((R, C), lambda i, j: (_out_band(i), j)),
        out_shape=jax.ShapeDtypeStruct((OUT, OUT), jnp.float32),
        scratch_shapes=[
            pltpu.VMEM((N, E), jnp.float32),
            pltpu.VMEM((1, OUT), jnp.float32),
            pltpu.SemaphoreType.DMA,
        ],
    )(M, p2, k2, s2)
    return out


# speedup vs baseline: 1.0165x; 1.0165x over previous
import jax
import jax.numpy as jnp
from jax.experimental import pallas as pl
from jax.experimental.pallas import tpu as pltpu

E = 2048
N = 1024
OUT = 2 * E + N
DT = 1e-06

R = 256
NB = OUT // R
N_KCL = N // R
N_KVL = E // R
N_EL = E // R


def _band_kernel(m_hbm, p_ref, k_ref, s_ref, out_ref, m_vmem, zy_vmem, sem):
    s = pl.program_id(0)

    @pl.when(s == 0)
    def _first_step():
        pltpu.make_async_copy(m_hbm, m_vmem, sem).start()
        params = p_ref[...]
        kinds = k_ref[...]
        sw_on = s_ref[...] > 0.0
        z = jnp.where(kinds == 0, -params,
            jnp.where(kinds == 4, -DT / params,
            jnp.where(kinds == 5, 1.0,
            jnp.where(kinds == 2, 1.0,
            jnp.where(jnp.logical_and(kinds == 3, jnp.logical_not(sw_on)),
                      1.0, 0.0)))))
        y = jnp.where(kinds == 0, 1.0,
            jnp.where(kinds == 4, 1.0,
            jnp.where(kinds == 5, -DT / params,
            jnp.where(kinds == 1, 1.0,
            jnp.where(jnp.logical_and(kinds == 3, sw_on), 1.0, 0.0)))))
        zy_vmem[:, 0:E] = z
        zy_vmem[:, E:2 * E] = y
        zy_vmem[:, 2 * E:] = jnp.zeros((1, N), jnp.float32)

    @pl.when(s == N_EL)
    def _wait_m_copy():
        pltpu.make_async_copy(m_hbm, m_vmem, sem).wait()

    def _zero_cols(lo, hi):
        if hi > lo:
            out_ref[:, lo:hi] = jnp.zeros((R, hi - lo), jnp.float32)

    def _strip(a, value):
        rs = jax.lax.broadcasted_iota(jnp.int32, (R, R), 0)
        cs = jax.lax.broadcasted_iota(jnp.int32, (R, R), 1)
        out_ref[:, a:a + R] = jnp.where(cs == rs, value, 0.0)

    def _el(e0):
        _zero_cols(0, e0)
        _strip(e0, zy_vmem[0:1, e0:e0 + R])
        _zero_cols(e0 + R, E + e0)
        _strip(E + e0, zy_vmem[0:1, E + e0:E + e0 + R])
        _zero_cols(E + e0 + R, OUT)

    def _kvl(e0):
        _zero_cols(0, E + e0)
        _strip(E + e0, 1.0)
        _zero_cols(E + e0 + R, 2 * E)
        out_ref[:, 2 * E:] = -m_vmem[:, e0:e0 + R].T

    def _kcl(r0):
        out_ref[:, 0:E] = m_vmem[r0:r0 + R, :]
        _zero_cols(E, OUT)

    for b in range(N_EL):
        pl.when(s == b)(lambda b=b: _el(b * R))
    for b in range(N_KVL):
        pl.when(s == N_EL + b)(lambda b=b: _kvl(b * R))
    for b in range(N_KCL):
        pl.when(s == N_EL + N_KVL + b)(lambda b=b: _kcl(b * R))


def _out_band(s):
    return jnp.where(s < N_EL, s + N_KCL + N_KVL,
           jnp.where(s < N_EL + N_KVL, s - N_EL + N_KCL,
                     s - N_EL - N_KVL))


def kernel(M, params, sw_params, kinds, time):
    swcol = sw_params[:, time]
    p2 = params.reshape(1, E).astype(jnp.float32)
    k2 = kinds.reshape(1, E).astype(jnp.int32)
    s2 = swcol.reshape(1, E).astype(jnp.float32)

    out = pl.pallas_call(
        _band_kernel,
        grid=(NB,),
        in_specs=[
            pl.BlockSpec(memory_space=pl.ANY),
            pl.BlockSpec((1, E), lambda i: (0, 0)),
            pl.BlockSpec((1, E), lambda i: (0, 0)),
            pl.BlockSpec((1, E), lambda i: (0, 0)),
        ],
        out_specs=pl.BlockSpec((R, OUT), lambda i: (_out_band(i), 0)),
        out_shape=jax.ShapeDtypeStruct((OUT, OUT), jnp.float32),
        scratch_shapes=[
            pltpu.VMEM((N, E), jnp.float32),
            pltpu.VMEM((1, OUT), jnp.float32),
            pltpu.SemaphoreType.DMA,
        ],
    )(M, p2, k2, s2)
    return out


# speedup vs baseline: 1.0214x; 1.0048x over previous
import jax
import jax.numpy as jnp
from jax.experimental import pallas as pl
from jax.experimental.pallas import tpu as pltpu

E = 2048
N = 1024
OUT = 2 * E + N
DT = 1e-06

R = 512
NB = OUT // R
N_KCL = N // R
N_KVL = E // R
N_EL = E // R


def _band_kernel(m_hbm, p_ref, k_ref, s_ref, out_ref, m_vmem, zy_vmem, sem):
    s = pl.program_id(0)

    @pl.when(s == 0)
    def _first_step():
        pltpu.make_async_copy(m_hbm, m_vmem, sem).start()
        params = p_ref[...]
        kinds = k_ref[...]
        sw_on = s_ref[...] > 0.0
        z = jnp.where(kinds == 0, -params,
            jnp.where(kinds == 4, -DT / params,
            jnp.where(kinds == 5, 1.0,
            jnp.where(kinds == 2, 1.0,
            jnp.where(jnp.logical_and(kinds == 3, jnp.logical_not(sw_on)),
                      1.0, 0.0)))))
        y = jnp.where(kinds == 0, 1.0,
            jnp.where(kinds == 4, 1.0,
            jnp.where(kinds == 5, -DT / params,
            jnp.where(kinds == 1, 1.0,
            jnp.where(jnp.logical_and(kinds == 3, sw_on), 1.0, 0.0)))))
        zy_vmem[:, 0:E] = z
        zy_vmem[:, E:2 * E] = y
        zy_vmem[:, 2 * E:] = jnp.zeros((1, N), jnp.float32)

    @pl.when(s == N_EL)
    def _wait_m_copy():
        pltpu.make_async_copy(m_hbm, m_vmem, sem).wait()

    def _zero_cols(lo, hi):
        if hi > lo:
            out_ref[:, lo:hi] = jnp.zeros((R, hi - lo), jnp.float32)

    def _strip(a, value):
        rs = jax.lax.broadcasted_iota(jnp.int32, (R, R), 0)
        cs = jax.lax.broadcasted_iota(jnp.int32, (R, R), 1)
        out_ref[:, a:a + R] = jnp.where(cs == rs, value, 0.0)

    def _el(e0):
        _zero_cols(0, e0)
        _strip(e0, zy_vmem[0:1, e0:e0 + R])
        _zero_cols(e0 + R, E + e0)
        _strip(E + e0, zy_vmem[0:1, E + e0:E + e0 + R])
        _zero_cols(E + e0 + R, OUT)

    def _kvl(e0):
        _zero_cols(0, E + e0)
        _strip(E + e0, 1.0)
        _zero_cols(E + e0 + R, 2 * E)
        out_ref[:, 2 * E:] = -m_vmem[:, e0:e0 + R].T

    def _kcl(r0):
        out_ref[:, 0:E] = m_vmem[r0:r0 + R, :]
        _zero_cols(E, OUT)

    for b in range(N_EL):
        pl.when(s == b)(lambda b=b: _el(b * R))
    for b in range(N_KVL):
        pl.when(s == N_EL + b)(lambda b=b: _kvl(b * R))
    for b in range(N_KCL):
        pl.when(s == N_EL + N_KVL + b)(lambda b=b: _kcl(b * R))


def _out_band(s):
    return jnp.where(s < N_EL, s + N_KCL + N_KVL,
           jnp.where(s < N_EL + N_KVL, s - N_EL + N_KCL,
                     s - N_EL - N_KVL))


def kernel(M, params, sw_params, kinds, time):
    swcol = sw_params[:, time]
    p2 = params.reshape(1, E).astype(jnp.float32)
    k2 = kinds.reshape(1, E).astype(jnp.int32)
    s2 = swcol.reshape(1, E).astype(jnp.float32)

    out = pl.pallas_call(
        _band_kernel,
        grid=(NB,),
        in_specs=[
            pl.BlockSpec(memory_space=pl.ANY),
            pl.BlockSpec((1, E), lambda i: (0, 0)),
            pl.BlockSpec((1, E), lambda i: (0, 0)),
            pl.BlockSpec((1, E), lambda i: (0, 0)),
        ],
        out_specs=pl.BlockSpec((R, OUT), lambda i: (_out_band(i), 0)),
        out_shape=jax.ShapeDtypeStruct((OUT, OUT), jnp.float32),
        scratch_shapes=[
            pltpu.VMEM((N, E), jnp.float32),
            pltpu.VMEM((1, OUT), jnp.float32),
            pltpu.SemaphoreType.DMA,
        ],
    )(M, p2, k2, s2)
    return out


# speedup vs baseline: 1.0226x; 1.0011x over previous
import jax
import jax.numpy as jnp
from jax.experimental import pallas as pl
from jax.experimental.pallas import tpu as pltpu

E = 2048
N = 1024
OUT = 2 * E + N
DT = 1e-06

R = 256
NB = OUT // R
N_KCL = N // R
N_KVL = E // R
N_EL = E // R
K = 4

_PERM = ([N_KCL + N_KVL + b for b in range(N_EL)]
         + [N_KCL + b for b in range(N_KVL)]
         + [b for b in range(N_KCL)])


def _band_kernel(m_hbm, p_ref, k_ref, s_ref, out_hbm,
                 m_vmem, zy_vmem, ring, m_sem, sems):
    s = pl.program_id(0)

    @pl.when(s == 0)
    def _first_step():
        pltpu.make_async_copy(m_hbm, m_vmem, m_sem).start()
        params = p_ref[...]
        kinds = k_ref[...]
        sw_on = s_ref[...] > 0.0
        z = jnp.where(kinds == 0, -params,
            jnp.where(kinds == 4, -DT / params,
            jnp.where(kinds == 5, 1.0,
            jnp.where(kinds == 2, 1.0,
            jnp.where(jnp.logical_and(kinds == 3, jnp.logical_not(sw_on)),
                      1.0, 0.0)))))
        y = jnp.where(kinds == 0, 1.0,
            jnp.where(kinds == 4, 1.0,
            jnp.where(kinds == 5, -DT / params,
            jnp.where(kinds == 1, 1.0,
            jnp.where(jnp.logical_and(kinds == 3, sw_on), 1.0, 0.0)))))
        zy_vmem[:, 0:E] = z
        zy_vmem[:, E:2 * E] = y
        zy_vmem[:, 2 * E:] = jnp.zeros((1, N), jnp.float32)

    @pl.when(s == N_EL)
    def _wait_m_copy():
        pltpu.make_async_copy(m_hbm, m_vmem, m_sem).wait()

    def _out_copy(b):
        return pltpu.make_async_copy(
            ring.at[b % K],
            out_hbm.at[pl.ds(_PERM[b] * R, R), :],
            sems.at[b % K])

    def _zero_cols(slot, lo, hi):
        if hi > lo:
            ring[slot, :, lo:hi] = jnp.zeros((R, hi - lo), jnp.float32)

    def _strip(slot, a, value):
        rs = jax.lax.broadcasted_iota(jnp.int32, (R, R), 0)
        cs = jax.lax.broadcasted_iota(jnp.int32, (R, R), 1)
        ring[slot, :, a:a + R] = jnp.where(cs == rs, value, 0.0)

    def _el(slot, e0):
        _zero_cols(slot, 0, e0)
        _strip(slot, e0, zy_vmem[0:1, e0:e0 + R])
        _zero_cols(slot, e0 + R, E + e0)
        _strip(slot, E + e0, zy_vmem[0:1, E + e0:E + e0 + R])
        _zero_cols(slot, E + e0 + R, OUT)

    def _kvl(slot, e0):
        _zero_cols(slot, 0, E + e0)
        _strip(slot, E + e0, 1.0)
        _zero_cols(slot, E + e0 + R, 2 * E)
        ring[slot, :, 2 * E:] = -m_vmem[:, e0:e0 + R].T

    def _kcl(slot, r0):
        ring[slot, :, 0:E] = m_vmem[r0:r0 + R, :]
        _zero_cols(slot, E, OUT)

    def _band_step(b):
        slot = b % K
        if b >= K:
            _out_copy(b - K).wait()
        if b < N_EL:
            _el(slot, b * R)
        elif b < N_EL + N_KVL:
            _kvl(slot, (b - N_EL) * R)
        else:
            _kcl(slot, (b - N_EL - N_KVL) * R)
        _out_copy(b).start()
        if b == NB - 1:
            for t in range(NB - K, NB):
                _out_copy(t).wait()

    for b in range(NB):
        pl.when(s == b)(lambda b=b: _band_step(b))


def kernel(M, params, sw_params, kinds, time):
    swcol = sw_params[:, time]
    p2 = params.reshape(1, E).astype(jnp.float32)
    k2 = kinds.reshape(1, E).astype(jnp.int32)
    s2 = swcol.reshape(1, E).astype(jnp.float32)

    out = pl.pallas_call(
        _band_kernel,
        grid=(NB,),
        in_specs=[
            pl.BlockSpec(memory_space=pl.ANY),
            pl.BlockSpec((1, E), lambda i: (0, 0)),
            pl.BlockSpec((1, E), lambda i: (0, 0)),
            pl.BlockSpec((1, E), lambda i: (0, 0)),
        ],
        out_specs=pl.BlockSpec(memory_space=pl.ANY),
        out_shape=jax.ShapeDtypeStruct((OUT, OUT), jnp.float32),
        scratch_shapes=[
            pltpu.VMEM((N, E), jnp.float32),
            pltpu.VMEM((1, OUT), jnp.float32),
            pltpu.VMEM((K, R, OUT), jnp.float32),
            pltpu.SemaphoreType.DMA,
            pltpu.SemaphoreType.DMA((K,)),
        ],
    )(M, p2, k2, s2)
    return out
